# Initial kernel scaffold; baseline (speedup 1.0000x reference)
import jax
import jax.numpy as jnp
from jax.experimental import pallas as pl


def kernel(node_feat, src, edge_type, edge_feat, edge_emb, W, b):
    raise NotImplementedError("write your pallas kernel here")


# speedup vs baseline: 2.2745x; 2.2745x over previous
import functools

import jax
import jax.numpy as jnp
from jax import lax
from jax.experimental import pallas as pl
from jax.experimental.pallas import tpu as pltpu
from jax.experimental.pallas import tpu_sc as plsc

EDGES = 320000
DIM = 128
NC = 2
NS = 16
NW = NC * NS
ROWS_W = EDGES // NW
CHUNK = 80
NCHUNK = ROWS_W // CHUNK


def _sc_gather(node_feat, edge_emb, src, edge_type):
  mesh = plsc.VectorSubcoreMesh(core_axis_name="c", subcore_axis_name="s")

  @functools.partial(
      pl.kernel,
      mesh=mesh,
      out_type=[
          jax.ShapeDtypeStruct((EDGES, DIM), jnp.float32),
          jax.ShapeDtypeStruct((EDGES, DIM), jnp.float32),
      ],
      scratch_types=[
          pltpu.VMEM((CHUNK,), jnp.int32),
          pltpu.VMEM((CHUNK,), jnp.int32),
          pltpu.VMEM((CHUNK, DIM), jnp.float32),
          pltpu.VMEM((CHUNK, DIM), jnp.float32),
          pltpu.SemaphoreType.DMA,
          pltpu.SemaphoreType.DMA,
      ],
  )
  def k(nf_hbm, emb_hbm, src_hbm, et_hbm, s1_hbm, s2_hbm,
        idx1, idx2, buf1, buf2, sem1, sem2):
    wid = lax.axis_index("s") * NC + lax.axis_index("c")
    base = wid * ROWS_W

    def body(c, carry):
      off = base + c * CHUNK
      pltpu.sync_copy(src_hbm.at[pl.ds(off, CHUNK)], idx1)
      pltpu.sync_copy(et_hbm.at[pl.ds(off, CHUNK)], idx2)
      cp1 = pltpu.async_copy(nf_hbm.at[idx1], buf1, sem1)
      cp2 = pltpu.async_copy(emb_hbm.at[idx2], buf2, sem2)
      cp1.wait()
      pltpu.sync_copy(buf1, s1_hbm.at[pl.ds(off, CHUNK)])
      cp2.wait()
      pltpu.sync_copy(buf2, s2_hbm.at[pl.ds(off, CHUNK)])
      return carry

    lax.fori_loop(0, NCHUNK, body, 0)

  return k(node_feat, edge_emb, src, edge_type)


BE = 1280
NB = EDGES // BE


def _tc_matmul(ef, s1, s2, W, b2):
  def body(ef_ref, s1_ref, s2_ref, w_ref, b_ref, o_ref):
    msg = ef_ref[...] + s1_ref[...] + s2_ref[...]
    acc = lax.dot_general(msg, w_ref[...], (((1,), (1,)), ((), ())),
                          preferred_element_type=jnp.float32)
    o_ref[...] = jnp.maximum(acc + b_ref[...], 0.0)

  return pl.pallas_call(
      body,
      grid=(NB,),
      in_specs=[
          pl.BlockSpec((BE, DIM), lambda i: (i, 0)),
          pl.BlockSpec((BE, DIM), lambda i: (i, 0)),
          pl.BlockSpec((BE, DIM), lambda i: (i, 0)),
          pl.BlockSpec((DIM, DIM), lambda i: (0, 0)),
          pl.BlockSpec((1, DIM), lambda i: (0, 0)),
      ],
      out_specs=pl.BlockSpec((BE, DIM), lambda i: (i, 0)),
      out_shape=jax.ShapeDtypeStruct((EDGES, DIM), jnp.float32),
  )(ef, s1, s2, W, b2)


def kernel(node_feat, src, edge_type, edge_feat, edge_emb, W, b):
  s1, s2 = _sc_gather(node_feat, edge_emb, src, edge_type)
  return _tc_matmul(edge_feat, s1, s2, W, b.reshape(1, DIM))


# speedup vs baseline: 3.3413x; 1.4690x over previous
import functools

import jax
import jax.numpy as jnp
from jax import lax
from jax.experimental import pallas as pl
from jax.experimental.pallas import tpu as pltpu
from jax.experimental.pallas import tpu_sc as plsc

EDGES = 320000
DIM = 128
NTYPES = 256
NC = 2
NS = 16
NW = NC * NS
ROWS_W = EDGES // NW
CHUNK = 80
NCHUNK = ROWS_W // CHUNK


def _sc_gather(node_feat, src):
  mesh = plsc.VectorSubcoreMesh(core_axis_name="c", subcore_axis_name="s")

  @functools.partial(
      pl.kernel,
      mesh=mesh,
      out_type=jax.ShapeDtypeStruct((EDGES, DIM), jnp.float32),
      scratch_types=[
          pltpu.VMEM((ROWS_W,), jnp.int32),
          pltpu.VMEM((CHUNK, DIM), jnp.float32),
          pltpu.VMEM((CHUNK, DIM), jnp.float32),
          pltpu.SemaphoreType.DMA,
          pltpu.SemaphoreType.DMA,
          pltpu.SemaphoreType.DMA,
          pltpu.SemaphoreType.DMA,
      ],
  )
  def k(nf_hbm, src_hbm, s1_hbm,
        idx_all, buf0, buf1, gsem0, gsem1, ssem0, ssem1):
    wid = lax.axis_index("s") * NC + lax.axis_index("c")
    base = wid * ROWS_W
    bufs = (buf0, buf1)
    gsems = (gsem0, gsem1)
    ssems = (ssem0, ssem1)

    pltpu.sync_copy(src_hbm.at[pl.ds(base, ROWS_W)], idx_all)

    def gfire(c, b):
      pltpu.async_copy(
          nf_hbm.at[idx_all.at[pl.ds(c * CHUNK, CHUNK)]], bufs[b], gsems[b])

    def gwait(c, b):
      pltpu.make_async_copy(
          nf_hbm.at[idx_all.at[pl.ds(c * CHUNK, CHUNK)]], bufs[b],
          gsems[b]).wait()

    def sfire(c, b):
      pltpu.async_copy(
          bufs[b], s1_hbm.at[pl.ds(base + c * CHUNK, CHUNK)], ssems[b])

    def swait(c, b):
      pltpu.make_async_copy(
          bufs[b], s1_hbm.at[pl.ds(base + c * CHUNK, CHUNK)],
          ssems[b]).wait()

    gfire(0, 0)
    gfire(1, 1)

    def body(i, carry):
      t = i * 2
      for b in (0, 1):
        c = t + b
        gwait(c, b)
        sfire(c, b)
        swait(c, b)
        gfire(c + 2, b)
      return carry

    lax.fori_loop(0, (NCHUNK - 3) // 2, body, 0)

    c = NCHUNK - 3
    gwait(c, 0); sfire(c, 0); swait(c, 0); gfire(c + 2, 0)
    c = NCHUNK - 2
    gwait(c, 1); sfire(c, 1); swait(c, 1)
    c = NCHUNK - 1
    gwait(c, 0); sfire(c, 0); swait(c, 0)

  return k(node_feat, src)


BE = 1280
NB = EDGES // BE


def _tc_matmul(et2, ef, s1, emb, W, b2):
  def body(et_ref, ef_ref, s1_ref, emb_ref, w_ref, b_ref, o_ref):
    onehot = (et_ref[...] == lax.broadcasted_iota(
        jnp.int32, (1, NTYPES), 1)).astype(jnp.float32)
    rel = lax.dot_general(onehot, emb_ref[...], (((1,), (0,)), ((), ())),
                          preferred_element_type=jnp.float32)
    msg = ef_ref[...] + s1_ref[...] + rel
    acc = lax.dot_general(msg, w_ref[...], (((1,), (1,)), ((), ())),
                          preferred_element_type=jnp.float32)
    o_ref[...] = jnp.maximum(acc + b_ref[...], 0.0)

  return pl.pallas_call(
      body,
      grid=(NB,),
      in_specs=[
          pl.BlockSpec((BE, 1), lambda i: (i, 0)),
          pl.BlockSpec((BE, DIM), lambda i: (i, 0)),
          pl.BlockSpec((BE, DIM), lambda i: (i, 0)),
          pl.BlockSpec((NTYPES, DIM), lambda i: (0, 0)),
          pl.BlockSpec((DIM, DIM), lambda i: (0, 0)),
          pl.BlockSpec((1, DIM), lambda i: (0, 0)),
      ],
      out_specs=pl.BlockSpec((BE, DIM), lambda i: (i, 0)),
      out_shape=jax.ShapeDtypeStruct((EDGES, DIM), jnp.float32),
  )(et2, ef, s1, emb, W, b2)


def kernel(node_feat, src, edge_type, edge_feat, edge_emb, W, b):
  s1 = _sc_gather(node_feat, src)
  return _tc_matmul(edge_type.reshape(EDGES, 1), edge_feat, s1,
                    edge_emb, W, b.reshape(1, DIM))


# speedup vs baseline: 3.3914x; 1.0150x over previous
import functools

import jax
import jax.numpy as jnp
from jax import lax
from jax.experimental import pallas as pl
from jax.experimental.pallas import tpu as pltpu
from jax.experimental.pallas import tpu_sc as plsc

EDGES = 320000
DIM = 128
NTYPES = 256
NC = 2
NS = 16
NW = NC * NS
NSLAB = 5
SLAB_E = EDGES // NSLAB
ROWS_W = SLAB_E // NW
CHUNK = 80
NCHUNK = ROWS_W // CHUNK

BE = 1280
NB_SLAB = SLAB_E // BE


def _sc_gather_slab(node_feat, src, slab):
  mesh = plsc.VectorSubcoreMesh(core_axis_name="c", subcore_axis_name="s")

  @functools.partial(
      pl.kernel,
      mesh=mesh,
      out_type=jax.ShapeDtypeStruct((SLAB_E, DIM), jnp.float32),
      scratch_types=[
          pltpu.VMEM((ROWS_W,), jnp.int32),
          pltpu.VMEM((CHUNK, DIM), jnp.float32),
          pltpu.VMEM((CHUNK, DIM), jnp.float32),
          pltpu.SemaphoreType.DMA,
          pltpu.SemaphoreType.DMA,
          pltpu.SemaphoreType.DMA,
          pltpu.SemaphoreType.DMA,
      ],
  )
  def k(nf_hbm, src_hbm, s1_hbm,
        idx_all, buf0, buf1, gsem0, gsem1, ssem0, ssem1):
    wid = lax.axis_index("s") * NC + lax.axis_index("c")
    lbase = wid * ROWS_W
    gbase = slab * SLAB_E + lbase
    bufs = (buf0, buf1)
    gsems = (gsem0, gsem1)
    ssems = (ssem0, ssem1)

    pltpu.sync_copy(src_hbm.at[pl.ds(gbase, ROWS_W)], idx_all)

    def gfire(c, b):
      pltpu.async_copy(
          nf_hbm.at[idx_all.at[pl.ds(c * CHUNK, CHUNK)]], bufs[b], gsems[b])

    def gwait(c, b):
      pltpu.make_async_copy(
          nf_hbm.at[idx_all.at[pl.ds(c * CHUNK, CHUNK)]], bufs[b],
          gsems[b]).wait()

    def sfire(c, b):
      pltpu.async_copy(
          bufs[b], s1_hbm.at[pl.ds(lbase + c * CHUNK, CHUNK)], ssems[b])

    def swait(c, b):
      pltpu.make_async_copy(
          bufs[b], s1_hbm.at[pl.ds(lbase + c * CHUNK, CHUNK)],
          ssems[b]).wait()

    gfire(0, 0)
    gfire(1, 1)

    def body(i, carry):
      t = i * 2
      for b in (0, 1):
        c = t + b
        gwait(c, b)
        sfire(c, b)
        swait(c, b)
        gfire(c + 2, b)
      return carry

    lax.fori_loop(0, (NCHUNK - 3) // 2, body, 0)

    c = NCHUNK - 3
    gwait(c, 0); sfire(c, 0); swait(c, 0); gfire(c + 2, 0)
    c = NCHUNK - 2
    gwait(c, 1); sfire(c, 1); swait(c, 1)
    c = NCHUNK - 1
    gwait(c, 0); sfire(c, 0); swait(c, 0)

  return k(node_feat, src)


def _tc_slab(h_acc, et2, ef, s1, emb, W, b2, slab):
  def body(*refs):
    if h_acc is None:
      et_ref, ef_ref, s1_ref, emb_ref, w_ref, b_ref, o_ref = refs
    else:
      _, et_ref, ef_ref, s1_ref, emb_ref, w_ref, b_ref, o_ref = refs
    onehot = (et_ref[...] == lax.broadcasted_iota(
        jnp.int32, (1, NTYPES), 1)).astype(jnp.float32)
    rel = lax.dot_general(onehot, emb_ref[...], (((1,), (0,)), ((), ())),
                          preferred_element_type=jnp.float32)
    msg = ef_ref[...] + s1_ref[...] + rel
    acc = lax.dot_general(msg, w_ref[...], (((1,), (1,)), ((), ())),
                          preferred_element_type=jnp.float32)
    o_ref[...] = jnp.maximum(acc + b_ref[...], 0.0)

  off = slab * NB_SLAB
  data_specs = [
      pl.BlockSpec((BE, 1), lambda i: (off + i, 0)),
      pl.BlockSpec((BE, DIM), lambda i: (off + i, 0)),
      pl.BlockSpec((BE, DIM), lambda i: (i, 0)),
      pl.BlockSpec((NTYPES, DIM), lambda i: (0, 0)),
      pl.BlockSpec((DIM, DIM), lambda i: (0, 0)),
      pl.BlockSpec((1, DIM), lambda i: (0, 0)),
  ]
  if h_acc is None:
    in_specs, aliases, args = data_specs, {}, (et2, ef, s1, emb, W, b2)
  else:
    in_specs = [pl.BlockSpec(memory_space=pltpu.MemorySpace.HBM)] + data_specs
    aliases = {0: 0}
    args = (h_acc, et2, ef, s1, emb, W, b2)
  return pl.pallas_call(
      body,
      grid=(NB_SLAB,),
      in_specs=in_specs,
      out_specs=pl.BlockSpec((BE, DIM), lambda i: (off + i, 0)),
      out_shape=jax.ShapeDtypeStruct((EDGES, DIM), jnp.float32),
      input_output_aliases=aliases,
  )(*args)


def kernel(node_feat, src, edge_type, edge_feat, edge_emb, W, b):
  et2 = edge_type.reshape(EDGES, 1)
  b2 = b.reshape(1, DIM)
  s1 = [_sc_gather_slab(node_feat, src, s) for s in range(NSLAB)]
  h = None
  for s in range(NSLAB):
    h = _tc_slab(h, et2, edge_feat, s1[s], edge_emb, W, b2, s)
  return h


# speedup vs baseline: 4.1073x; 1.2111x over previous
import functools

import jax
import jax.numpy as jnp
from jax import lax
from jax.experimental import pallas as pl
from jax.experimental.pallas import tpu as pltpu
from jax.experimental.pallas import tpu_sc as plsc

EDGES = 320000
DIM = 128
NTYPES = 256
NC = 2
NS = 16
NW = NC * NS
NSLAB = 5
SLAB_E = EDGES // NSLAB
ROWS_W = SLAB_E // NW
CHUNK = 80
NCHUNK = ROWS_W // CHUNK

BE = 3200
NB_SLAB = SLAB_E // BE


def _sc_gather_slab(node_feat, src, slab):
  mesh = plsc.VectorSubcoreMesh(core_axis_name="c", subcore_axis_name="s")

  @functools.partial(
      pl.kernel,
      mesh=mesh,
      out_type=jax.ShapeDtypeStruct((SLAB_E, DIM), jnp.float32),
      scratch_types=[
          pltpu.VMEM((ROWS_W,), jnp.int32),
          pltpu.VMEM((CHUNK, DIM), jnp.float32),
          pltpu.VMEM((CHUNK, DIM), jnp.float32),
          pltpu.SemaphoreType.DMA,
          pltpu.SemaphoreType.DMA,
          pltpu.SemaphoreType.DMA,
          pltpu.SemaphoreType.DMA,
      ],
  )
  def k(nf_hbm, src_hbm, s1_hbm,
        idx_all, buf0, buf1, gsem0, gsem1, ssem0, ssem1):
    wid = lax.axis_index("s") * NC + lax.axis_index("c")
    lbase = wid * ROWS_W
    gbase = slab * SLAB_E + lbase
    bufs = (buf0, buf1)
    gsems = (gsem0, gsem1)
    ssems = (ssem0, ssem1)

    pltpu.sync_copy(src_hbm.at[pl.ds(gbase, ROWS_W)], idx_all)

    def gfire(c, b):
      pltpu.async_copy(
          nf_hbm.at[idx_all.at[pl.ds(c * CHUNK, CHUNK)]], bufs[b], gsems[b])

    def gwait(c, b):
      pltpu.make_async_copy(
          nf_hbm.at[idx_all.at[pl.ds(c * CHUNK, CHUNK)]], bufs[b],
          gsems[b]).wait()

    def sfire(c, b):
      pltpu.async_copy(
          bufs[b], s1_hbm.at[pl.ds(lbase + c * CHUNK, CHUNK)], ssems[b])

    def swait(c, b):
      pltpu.make_async_copy(
          bufs[b], s1_hbm.at[pl.ds(lbase + c * CHUNK, CHUNK)],
          ssems[b]).wait()

    gfire(0, 0)
    gfire(1, 1)

    def body(i, carry):
      t = i * 2
      for b in (0, 1):
        c = t + b
        gwait(c, b)
        sfire(c, b)
        swait(c, b)
        gfire(c + 2, b)
      return carry

    lax.fori_loop(0, (NCHUNK - 3) // 2, body, 0)

    c = NCHUNK - 3
    gwait(c, 0); sfire(c, 0); swait(c, 0); gfire(c + 2, 0)
    c = NCHUNK - 2
    gwait(c, 1); sfire(c, 1); swait(c, 1)
    c = NCHUNK - 1
    gwait(c, 0); sfire(c, 0); swait(c, 0)

  return k(node_feat, src)


def _tc_slab(h_acc, et2, ef, s1, emb, W, b2, slab):
  def body(*refs):
    if h_acc is None:
      et_ref, ef_ref, s1_ref, emb_ref, w_ref, b_ref, o_ref = refs
    else:
      _, et_ref, ef_ref, s1_ref, emb_ref, w_ref, b_ref, o_ref = refs
    onehot = (et_ref[...] == lax.broadcasted_iota(
        jnp.int32, (1, NTYPES), 1)).astype(jnp.float32)
    rel = lax.dot_general(onehot, emb_ref[...], (((1,), (0,)), ((), ())),
                          preferred_element_type=jnp.float32)
    msg = ef_ref[...] + s1_ref[...] + rel
    acc = lax.dot_general(msg, w_ref[...], (((1,), (1,)), ((), ())),
                          preferred_element_type=jnp.float32)
    o_ref[...] = jnp.maximum(acc + b_ref[...], 0.0)

  off = slab * NB_SLAB
  data_specs = [
      pl.BlockSpec((BE, 1), lambda i: (off + i, 0)),
      pl.BlockSpec((BE, DIM), lambda i: (off + i, 0)),
      pl.BlockSpec((BE, DIM), lambda i: (i, 0)),
      pl.BlockSpec((NTYPES, DIM), lambda i: (0, 0)),
      pl.BlockSpec((DIM, DIM), lambda i: (0, 0)),
      pl.BlockSpec((1, DIM), lambda i: (0, 0)),
  ]
  if h_acc is None:
    in_specs, aliases, args = data_specs, {}, (et2, ef, s1, emb, W, b2)
  else:
    in_specs = [pl.BlockSpec(memory_space=pltpu.MemorySpace.HBM)] + data_specs
    aliases = {0: 0}
    args = (h_acc, et2, ef, s1, emb, W, b2)
  return pl.pallas_call(
      body,
      grid=(NB_SLAB,),
      in_specs=in_specs,
      out_specs=pl.BlockSpec((BE, DIM), lambda i: (off + i, 0)),
      out_shape=jax.ShapeDtypeStruct((EDGES, DIM), jnp.float32),
      input_output_aliases=aliases,
  )(*args)


def kernel(node_feat, src, edge_type, edge_feat, edge_emb, W, b):
  et2 = edge_type.reshape(EDGES, 1)
  b2 = b.reshape(1, DIM)
  s1 = [_sc_gather_slab(node_feat, src, s) for s in range(NSLAB)]
  h = None
  for s in range(NSLAB):
    h = _tc_slab(h, et2, edge_feat, s1[s], edge_emb, W, b2, s)
  return h


# speedup vs baseline: 4.2380x; 1.0318x over previous
import functools

import jax
import jax.numpy as jnp
from jax import lax
from jax.experimental import pallas as pl
from jax.experimental.pallas import tpu as pltpu
from jax.experimental.pallas import tpu_sc as plsc

EDGES = 320000
DIM = 128
NTYPES = 256
NC = 2
NS = 16
NW = NC * NS
NSLAB = 5
SLAB_E = EDGES // NSLAB
ROWS_W = SLAB_E // NW
CHUNK = 80
NCHUNK = ROWS_W // CHUNK

BE = 6400
NB_SLAB = SLAB_E // BE


def _sc_gather_slab(node_feat, src, slab):
  mesh = plsc.VectorSubcoreMesh(core_axis_name="c", subcore_axis_name="s")

  @functools.partial(
      pl.kernel,
      mesh=mesh,
      out_type=jax.ShapeDtypeStruct((SLAB_E, DIM), jnp.float32),
      scratch_types=[
          pltpu.VMEM((ROWS_W,), jnp.int32),
          pltpu.VMEM((CHUNK, DIM), jnp.float32),
          pltpu.VMEM((CHUNK, DIM), jnp.float32),
          pltpu.SemaphoreType.DMA,
          pltpu.SemaphoreType.DMA,
          pltpu.SemaphoreType.DMA,
          pltpu.SemaphoreType.DMA,
      ],
  )
  def k(nf_hbm, src_hbm, s1_hbm,
        idx_all, buf0, buf1, gsem0, gsem1, ssem0, ssem1):
    wid = lax.axis_index("s") * NC + lax.axis_index("c")
    lbase = wid * ROWS_W
    gbase = slab * SLAB_E + lbase
    bufs = (buf0, buf1)
    gsems = (gsem0, gsem1)
    ssems = (ssem0, ssem1)

    pltpu.sync_copy(src_hbm.at[pl.ds(gbase, ROWS_W)], idx_all)

    def gfire(c, b):
      pltpu.async_copy(
          nf_hbm.at[idx_all.at[pl.ds(c * CHUNK, CHUNK)]], bufs[b], gsems[b])

    def gwait(c, b):
      pltpu.make_async_copy(
          nf_hbm.at[idx_all.at[pl.ds(c * CHUNK, CHUNK)]], bufs[b],
          gsems[b]).wait()

    def sfire(c, b):
      pltpu.async_copy(
          bufs[b], s1_hbm.at[pl.ds(lbase + c * CHUNK, CHUNK)], ssems[b])

    def swait(c, b):
      pltpu.make_async_copy(
          bufs[b], s1_hbm.at[pl.ds(lbase + c * CHUNK, CHUNK)],
          ssems[b]).wait()

    gfire(0, 0)
    gfire(1, 1)

    def body(i, carry):
      t = i * 2
      for b in (0, 1):
        c = t + b
        gwait(c, b)
        sfire(c, b)
        swait(c, b)
        gfire(c + 2, b)
      return carry

    lax.fori_loop(0, (NCHUNK - 3) // 2, body, 0)

    c = NCHUNK - 3
    gwait(c, 0); sfire(c, 0); swait(c, 0); gfire(c + 2, 0)
    c = NCHUNK - 2
    gwait(c, 1); sfire(c, 1); swait(c, 1)
    c = NCHUNK - 1
    gwait(c, 0); sfire(c, 0); swait(c, 0)

  return k(node_feat, src)


def _tc_slab(h_acc, et2, ef, s1, emb, W, b2, slab):
  def body(*refs):
    if h_acc is None:
      et_ref, ef_ref, s1_ref, emb_ref, w_ref, b_ref, o_ref = refs
    else:
      _, et_ref, ef_ref, s1_ref, emb_ref, w_ref, b_ref, o_ref = refs
    onehot = (et_ref[...] == lax.broadcasted_iota(
        jnp.int32, (1, NTYPES), 1)).astype(jnp.float32)
    rel = lax.dot_general(onehot, emb_ref[...], (((1,), (0,)), ((), ())),
                          preferred_element_type=jnp.float32)
    msg = ef_ref[...] + s1_ref[...] + rel
    acc = lax.dot_general(msg, w_ref[...], (((1,), (1,)), ((), ())),
                          preferred_element_type=jnp.float32)
    o_ref[...] = jnp.maximum(acc + b_ref[...], 0.0)

  off = slab * NB_SLAB
  data_specs = [
      pl.BlockSpec((BE, 1), lambda i: (off + i, 0)),
      pl.BlockSpec((BE, DIM), lambda i: (off + i, 0)),
      pl.BlockSpec((BE, DIM), lambda i: (i, 0)),
      pl.BlockSpec((NTYPES, DIM), lambda i: (0, 0)),
      pl.BlockSpec((DIM, DIM), lambda i: (0, 0)),
      pl.BlockSpec((1, DIM), lambda i: (0, 0)),
  ]
  if h_acc is None:
    in_specs, aliases, args = data_specs, {}, (et2, ef, s1, emb, W, b2)
  else:
    in_specs = [pl.BlockSpec(memory_space=pltpu.MemorySpace.HBM)] + data_specs
    aliases = {0: 0}
    args = (h_acc, et2, ef, s1, emb, W, b2)
  return pl.pallas_call(
      body,
      grid=(NB_SLAB,),
      in_specs=in_specs,
      out_specs=pl.BlockSpec((BE, DIM), lambda i: (off + i, 0)),
      out_shape=jax.ShapeDtypeStruct((EDGES, DIM), jnp.float32),
      input_output_aliases=aliases,
  )(*args)


def kernel(node_feat, src, edge_type, edge_feat, edge_emb, W, b):
  et2 = edge_type.reshape(EDGES, 1)
  b2 = b.reshape(1, DIM)
  s1 = [_sc_gather_slab(node_feat, src, s) for s in range(NSLAB)]
  h = None
  for s in range(NSLAB):
    h = _tc_slab(h, et2, edge_feat, s1[s], edge_emb, W, b2, s)
  return h


# speedup vs baseline: 4.4140x; 1.0415x over previous
import functools

import jax
import jax.numpy as jnp
from jax import lax
from jax.experimental import pallas as pl
from jax.experimental.pallas import tpu as pltpu
from jax.experimental.pallas import tpu_sc as plsc

EDGES = 320000
DIM = 128
NTYPES = 256
NC = 2
NS = 16
NW = NC * NS
NSLAB = 5
SLAB_E = EDGES // NSLAB
HALF = SLAB_E // 2
PW = HALF // NW
CHUNKP = 40
NCHP = PW // CHUNKP

BE = 6400
NB_SLAB = SLAB_E // BE
NB_HALF = HALF // BE


def _sc_gather_slab(node_feat, src, slab):
  mesh = plsc.VectorSubcoreMesh(core_axis_name="c", subcore_axis_name="s")

  @functools.partial(
      pl.kernel,
      mesh=mesh,
      compiler_params=pltpu.CompilerParams(needs_layout_passes=False),
      out_type=jax.ShapeDtypeStruct((HALF, DIM), jnp.float32),
      scratch_types=[
          pltpu.VMEM((PW,), jnp.int32),
          pltpu.VMEM((PW,), jnp.int32),
          pltpu.VMEM((CHUNKP, DIM), jnp.float32),
          pltpu.VMEM((CHUNKP, DIM), jnp.float32),
          pltpu.VMEM((CHUNKP, DIM), jnp.float32),
          pltpu.VMEM((CHUNKP, DIM), jnp.float32),
          pltpu.VMEM((CHUNKP, DIM), jnp.float32),
          pltpu.VMEM((CHUNKP, DIM), jnp.float32),
          pltpu.SemaphoreType.DMA,
          pltpu.SemaphoreType.DMA,
          pltpu.SemaphoreType.DMA,
          pltpu.SemaphoreType.DMA,
          pltpu.SemaphoreType.DMA,
          pltpu.SemaphoreType.DMA,
      ],
  )
  def k(nf_hbm, src_hbm, s1_hbm,
        idxA, idxB, bufA0, bufA1, bufB0, bufB1, pbuf0, pbuf1,
        gsemA0, gsemA1, gsemB0, gsemB1, ssem0, ssem1):
    wid = lax.axis_index("s") * NC + lax.axis_index("c")
    wbase = wid * PW
    abase = slab * SLAB_E + wid * PW
    bbase = abase + HALF
    bufA = (bufA0, bufA1)
    bufB = (bufB0, bufB1)
    pbuf = (pbuf0, pbuf1)
    gsemA = (gsemA0, gsemA1)
    gsemB = (gsemB0, gsemB1)
    ssems = (ssem0, ssem1)

    pltpu.sync_copy(src_hbm.at[pl.ds(abase, PW)], idxA)
    pltpu.sync_copy(src_hbm.at[pl.ds(bbase, PW)], idxB)

    def gfire(c, b):
      pltpu.async_copy(
          nf_hbm.at[idxA.at[pl.ds(c * CHUNKP, CHUNKP)]], bufA[b], gsemA[b])
      pltpu.async_copy(
          nf_hbm.at[idxB.at[pl.ds(c * CHUNKP, CHUNKP)]], bufB[b], gsemB[b])

    def gwait(c, b):
      pltpu.make_async_copy(
          nf_hbm.at[idxA.at[pl.ds(c * CHUNKP, CHUNKP)]], bufA[b],
          gsemA[b]).wait()
      pltpu.make_async_copy(
          nf_hbm.at[idxB.at[pl.ds(c * CHUNKP, CHUNKP)]], bufB[b],
          gsemB[b]).wait()

    def pack_chunk(b):
      ba, bb, pb = bufA[b], bufB[b], pbuf[b]

      def prow(v, carry):
        for g in range(DIM // 16):
          sl = pl.ds(g * 16, 16)
          packed = plsc.pack(ba[v, sl], bb[v, sl],
                             format=plsc.PackFormat.INTERLEAVED)
          pb[v, sl] = plsc.bitcast(packed, jnp.float32)
        return carry

      lax.fori_loop(0, CHUNKP, prow, 0)

    def sfire(c, b):
      pltpu.async_copy(
          pbuf[b], s1_hbm.at[pl.ds(wbase + c * CHUNKP, CHUNKP)], ssems[b])

    def swait(c, b):
      pltpu.make_async_copy(
          pbuf[b], s1_hbm.at[pl.ds(wbase + c * CHUNKP, CHUNKP)],
          ssems[b]).wait()

    gfire(0, 0)
    gfire(1, 1)

    def body(i, carry):
      t = i * 2
      for b in (0, 1):
        c = t + b
        gwait(c, b)
        pack_chunk(b)
        sfire(c, b)
        gfire(c + 2, b)
        swait(c, b)
      return carry

    lax.fori_loop(0, (NCHP - 3) // 2, body, 0)

    c = NCHP - 3
    gwait(c, 0); pack_chunk(0); sfire(c, 0); gfire(c + 2, 0); swait(c, 0)
    c = NCHP - 2
    gwait(c, 1); pack_chunk(1); sfire(c, 1); swait(c, 1)
    c = NCHP - 1
    gwait(c, 0); pack_chunk(0); sfire(c, 0); swait(c, 0)

  return k(node_feat, src)


def _tc_slab(h_acc, et2, ef, s1p, emb, W, b2, slab):
  def body(*refs):
    if h_acc is None:
      et_ref, ef_ref, s1_ref, emb_ref, w_ref, b_ref, o_ref = refs
    else:
      _, et_ref, ef_ref, s1_ref, emb_ref, w_ref, b_ref, o_ref = refs
    words = lax.bitcast_convert_type(s1_ref[...], jnp.int32)
    is_high = pl.program_id(0) >= NB_HALF
    sel = jnp.where(is_high, words & jnp.int32(-65536),
                    words << 16)
    s1 = lax.bitcast_convert_type(sel, jnp.float32)
    onehot = (et_ref[...] == lax.broadcasted_iota(
        jnp.int32, (1, NTYPES), 1)).astype(jnp.float32)
    rel = lax.dot_general(onehot, emb_ref[...], (((1,), (0,)), ((), ())),
                          preferred_element_type=jnp.float32)
    msg = ef_ref[...] + s1 + rel
    acc = lax.dot_general(msg, w_ref[...], (((1,), (1,)), ((), ())),
                          preferred_element_type=jnp.float32)
    o_ref[...] = jnp.maximum(acc + b_ref[...], 0.0)

  off = slab * NB_SLAB
  data_specs = [
      pl.BlockSpec((BE, 1), lambda i: (off + i, 0)),
      pl.BlockSpec((BE, DIM), lambda i: (off + i, 0)),
      pl.BlockSpec((BE, DIM), lambda i: (lax.rem(i, NB_HALF), 0)),
      pl.BlockSpec((NTYPES, DIM), lambda i: (0, 0)),
      pl.BlockSpec((DIM, DIM), lambda i: (0, 0)),
      pl.BlockSpec((1, DIM), lambda i: (0, 0)),
  ]
  if h_acc is None:
    in_specs, aliases, args = data_specs, {}, (et2, ef, s1p, emb, W, b2)
  else:
    in_specs = [pl.BlockSpec(memory_space=pltpu.MemorySpace.HBM)] + data_specs
    aliases = {0: 0}
    args = (h_acc, et2, ef, s1p, emb, W, b2)
  return pl.pallas_call(
      body,
      grid=(NB_SLAB,),
      in_specs=in_specs,
      out_specs=pl.BlockSpec((BE, DIM), lambda i: (off + i, 0)),
      out_shape=jax.ShapeDtypeStruct((EDGES, DIM), jnp.float32),
      input_output_aliases=aliases,
  )(*args)


def kernel(node_feat, src, edge_type, edge_feat, edge_emb, W, b):
  et2 = edge_type.reshape(EDGES, 1)
  b2 = b.reshape(1, DIM)
  s1p = [_sc_gather_slab(node_feat, src, s) for s in range(NSLAB)]
  h = None
  for s in range(NSLAB):
    h = _tc_slab(h, et2, edge_feat, s1p[s], edge_emb, W, b2, s)
  return h
